# baseline (speedup 1.0000x reference)
import jax
import jax.numpy as jnp
from jax.experimental import pallas as pl

_N = 10000
_F_IN = 78
_HEADS = 10
_OUT_DIM = 128
_G = 32
_BLK = 1000


def _layer1_kernel(x_ref, w_ref, asrc_ref, adst_ref, h_ref, as_ref, ad_ref):
    h = x_ref[...] @ w_ref[...]
    h_ref[...] = h
    as_ref[...] = h @ asrc_ref[...]
    ad_ref[...] = h @ adst_ref[...]


def _layer2_kernel(agg_ref, b_ref, w_ref, asrc_ref, adst_ref, h_ref, as_ref, ad_ref):
    v = agg_ref[...] + b_ref[...]
    helu = jnp.where(v > 0, v, jnp.exp(jnp.minimum(v, 0.0)) - 1.0)
    h = helu @ w_ref[...]
    h_ref[...] = h
    as_ref[...] = h @ asrc_ref[...]
    ad_ref[...] = h @ adst_ref[...]


def _fc_kernel(p_ref, w_ref, b_ref, o_ref):
    o_ref[...] = jax.nn.relu(p_ref[...] @ w_ref[...] + b_ref[...])


def _dense_stage(kern, ins, in_specs, feat_dims):
    out_specs = [pl.BlockSpec((_BLK, d), lambda i: (i, 0)) for d in feat_dims]
    out_shape = [jax.ShapeDtypeStruct((_N, d), jnp.float32) for d in feat_dims]
    return pl.pallas_call(
        kern,
        grid=(_N // _BLK,),
        in_specs=in_specs,
        out_specs=out_specs,
        out_shape=out_shape,
    )(*ins)


def _edge_softmax_agg(h3, a_src, a_dst, src, dst):
    n, heads, d = h3.shape
    alpha = jax.nn.leaky_relu(a_src[src] + a_dst[dst], negative_slope=0.2)
    amax = jax.ops.segment_max(alpha, dst, num_segments=n)
    amax = jnp.where(jnp.isfinite(amax), amax, 0.0)
    alpha = jnp.exp(alpha - amax[dst])
    denom = jax.ops.segment_sum(alpha, dst, num_segments=n)
    alpha = alpha / (denom[dst] + 1e-16)
    msg = h3[src] * alpha[:, :, None]
    return jax.ops.segment_sum(msg, dst, num_segments=n)


def kernel(x, edge_index, batch, W1, asrc1, adst1, b1, W2, asrc2, adst2, b2, Wfc, bfc):
    ar = jnp.arange(_N, dtype=edge_index.dtype)
    src = jnp.concatenate([edge_index[0], ar])
    dst = jnp.concatenate([edge_index[1], ar])

    eye = jnp.eye(_HEADS, dtype=jnp.float32)
    Asrc1 = (asrc1[0][:, :, None] * eye[:, None, :]).reshape(_HEADS * _F_IN, _HEADS)
    Adst1 = (adst1[0][:, :, None] * eye[:, None, :]).reshape(_HEADS * _F_IN, _HEADS)
    Asrc2 = asrc2[0].T
    Adst2 = adst2[0].T

    hf = _HEADS * _F_IN

    h1, as1, ad1 = _dense_stage(
        _layer1_kernel,
        (x, W1, Asrc1, Adst1),
        [
            pl.BlockSpec((_BLK, _F_IN), lambda i: (i, 0)),
            pl.BlockSpec((_F_IN, hf), lambda i: (0, 0)),
            pl.BlockSpec((hf, _HEADS), lambda i: (0, 0)),
            pl.BlockSpec((hf, _HEADS), lambda i: (0, 0)),
        ],
        [hf, _HEADS, _HEADS],
    )

    agg1 = _edge_softmax_agg(h1.reshape(_N, _HEADS, _F_IN), as1, ad1, src, dst)
    agg1 = agg1.reshape(_N, hf)

    h2, as2, ad2 = _dense_stage(
        _layer2_kernel,
        (agg1, b1[None, :], W2, Asrc2, Adst2),
        [
            pl.BlockSpec((_BLK, hf), lambda i: (i, 0)),
            pl.BlockSpec((1, hf), lambda i: (0, 0)),
            pl.BlockSpec((hf, _OUT_DIM), lambda i: (0, 0)),
            pl.BlockSpec((_OUT_DIM, 1), lambda i: (0, 0)),
            pl.BlockSpec((_OUT_DIM, 1), lambda i: (0, 0)),
        ],
        [_OUT_DIM, 1, 1],
    )

    agg2 = _edge_softmax_agg(h2[:, None, :], as2, ad2, src, dst)[:, 0, :]

    h = jax.nn.relu(agg2 + b2)
    pooled = jax.ops.segment_max(h, batch, num_segments=_G)
    pooled = jnp.where(jnp.isfinite(pooled), pooled, 0.0)

    out = pl.pallas_call(
        _fc_kernel,
        out_shape=jax.ShapeDtypeStruct((_G, _OUT_DIM), jnp.float32),
    )(pooled, Wfc, bfc[None, :])
    return out
